# baseline (speedup 1.0000x reference)
import jax
import jax.numpy as jnp
from jax import lax
from jax.experimental import pallas as pl
from jax.experimental.pallas import tpu as pltpu

_B, _N, _D = 4, 5000, 256
_NP = 5120
_JB = 128
_KB = 512
_IB = 512
_NB = _NP // _JB
_NEG = -1e30
_IOU_T = 0.5
_TH = 0.2
_KMIN, _KMAX = 3, 15


def _body(sz_ref, br_ref, bc_ref, hs_ref, obox_ref, oaux_ref, ohs_ref,
          A_ref, OH_ref, KP_ref):
    pid = pl.program_id(0)
    szh = sz_ref[pid, 0]
    szw = sz_ref[pid, 1]

    x1c = bc_ref[0, :, 0:1]
    y1c = bc_ref[0, :, 1:2]
    x2c = bc_ref[0, :, 2:3]
    y2c = bc_ref[0, :, 3:4]
    scc = bc_ref[0, :, 4:5]
    lbc = bc_ref[0, :, 5:6]

    maxc = jnp.max(br_ref[0, 0:4, :]) + 1.0

    iotar = lax.broadcasted_iota(jnp.int32, (1, _NP), 1)

    def jloop(jb, _):
        j0 = pl.multiple_of(jb * _JB, _JB)
        blb = br_ref[0, 5:6, pl.ds(j0, _JB)]
        bx1 = br_ref[0, 0:1, pl.ds(j0, _JB)] + blb * maxc
        by1 = br_ref[0, 1:2, pl.ds(j0, _JB)] + blb * maxc
        bx2 = br_ref[0, 2:3, pl.ds(j0, _JB)] + blb * maxc
        by2 = br_ref[0, 3:4, pl.ds(j0, _JB)] + blb * maxc
        bar = jnp.maximum(bx2 - bx1, 0.0) * jnp.maximum(by2 - by1, 0.0)
        bsc = br_ref[0, 4:5, pl.ds(j0, _JB)]
        bio = j0 + lax.broadcasted_iota(jnp.int32, (1, _JB), 1)

        def iloop(ib, _2):
            i0 = pl.multiple_of(ib * _IB, _IB)
            clb = bc_ref[0, pl.ds(i0, _IB), 5:6]
            cx1 = bc_ref[0, pl.ds(i0, _IB), 0:1] + clb * maxc
            cy1 = bc_ref[0, pl.ds(i0, _IB), 1:2] + clb * maxc
            cx2 = bc_ref[0, pl.ds(i0, _IB), 2:3] + clb * maxc
            cy2 = bc_ref[0, pl.ds(i0, _IB), 3:4] + clb * maxc
            car = (jnp.maximum(cx2 - cx1, 0.0) *
                   jnp.maximum(cy2 - cy1, 0.0))
            csc = bc_ref[0, pl.ds(i0, _IB), 4:5]
            cio = i0 + lax.broadcasted_iota(jnp.int32, (_IB, 1), 0)
            iw = jnp.maximum(
                jnp.minimum(cx2, bx2) - jnp.maximum(cx1, bx1), 0.0)
            ih = jnp.maximum(
                jnp.minimum(cy2, by2) - jnp.maximum(cy1, by1), 0.0)
            inter = iw * ih
            iou = inter / (car + bar - inter + 1e-9)
            prio = (csc > bsc) | ((csc == bsc) & (cio < bio))
            sup = jnp.where((iou > _IOU_T) & prio, 1, 0).astype(jnp.int8)
            A_ref[pl.ds(i0, _IB), pl.ds(j0, _JB)] = sup
            return 0

        lax.fori_loop(0, _NP // _IB, iloop, 0)
        return 0

    lax.fori_loop(0, _NB, jloop, 0)

    def fp_cond(c):
        return c[1]

    def fp_body(c):
        keep, _ = c
        KP_ref[...] = keep.astype(jnp.int8)

        def acc(kb, s):
            k0 = pl.multiple_of(kb * _KB, _KB)
            return s + jnp.dot(KP_ref[0:1, pl.ds(k0, _KB)],
                               A_ref[pl.ds(k0, _KB), :],
                               preferred_element_type=jnp.int32)

        supn = lax.fori_loop(0, _NP // _KB, acc,
                             jnp.zeros((1, _NP), dtype=jnp.int32))
        keep_new = jnp.where(supn == 0, 1, 0)
        ndiff = jnp.sum(jnp.abs(keep_new - keep))
        return keep_new, ndiff > 0

    keep, _ = lax.while_loop(
        fp_cond, fp_body,
        (jnp.ones((1, _NP), dtype=jnp.int32), True))
    kb = keep != 0

    scr = br_ref[0, 4:5, :]
    lbr = br_ref[0, 5:6, :]
    is_h = lbr == 0.0
    h_sc0 = jnp.where(kb & is_h, scr, _NEG)
    o_sc0 = jnp.where(kb & (~is_h), scr, _NEG)
    n_h = jnp.sum(jnp.where(h_sc0 >= _TH, 1, 0))
    n_o = jnp.sum(jnp.where(o_sc0 >= _TH, 1, 0))
    k_h = jnp.clip(n_h, _KMIN, _KMAX)
    k_o = jnp.clip(n_o, _KMIN, _KMAX)

    OH_ref[...] = jnp.zeros((32, _NP), dtype=jnp.float32)
    rows32 = lax.broadcasted_iota(jnp.int32, (32, 1), 0)

    def pick(r, c):
        h, o, vvec = c
        cur = jnp.where(r < _KMAX, h, o)
        m = jnp.max(cur)
        selidx = jnp.min(jnp.where(cur == m, iotar, _NP))
        OH_ref[pl.ds(r, 1), :] = jnp.where(iotar == selidx, 1.0, 0.0)
        vvec = jnp.where(rows32 == r, m, vvec)
        hit = iotar == selidx
        h = jnp.where(hit & (r < _KMAX), -jnp.inf, h)
        o = jnp.where(hit & (r >= _KMAX), -jnp.inf, o)
        return h, o, vvec

    _, _, vvec = lax.fori_loop(
        0, 2 * _KMAX, pick,
        (h_sc0, o_sc0, jnp.full((32, 1), -jnp.inf, dtype=jnp.float32)))

    limit = jnp.where(rows32 < _KMAX, k_h, _KMAX + k_o)
    mask = (rows32 < limit) & (vvec > _NEG * 0.5)
    mf = jnp.where(mask, 1.0, 0.0)

    bxcol = jnp.concatenate(
        [jnp.maximum(x1c, 0.0), jnp.maximum(y1c, 0.0),
         jnp.minimum(x2c, szw), jnp.minimum(y2c, szh)], axis=1)
    oh = OH_ref[...]
    hp = lax.Precision.HIGHEST
    gbox = jnp.dot(oh, bxcol, precision=hp,
                   preferred_element_type=jnp.float32) * mf
    gsc = jnp.dot(oh, scc, precision=hp,
                  preferred_element_type=jnp.float32) * mf
    glb = jnp.dot(oh, lbc, precision=hp,
                  preferred_element_type=jnp.float32)
    ghs = jnp.dot(oh, hs_ref[0], precision=hp,
                  preferred_element_type=jnp.float32) * mf
    lbout = jnp.where(mask, glb, -1.0)

    obox_ref[0] = gbox
    oaux_ref[0] = jnp.concatenate(
        [gsc, lbout, mf, jnp.zeros((32, 1), dtype=jnp.float32)], axis=1)
    ohs_ref[0] = ghs


def kernel(boxes, scores, hidden_states, labels, image_sizes):
    pad = _NP - _N
    sc_p = jnp.pad(scores.astype(jnp.float32), ((0, 0), (0, pad)),
                   constant_values=_NEG)
    bx_p = jnp.pad(boxes.astype(jnp.float32), ((0, 0), (0, pad), (0, 0)))
    lb_p = jnp.pad(labels.astype(jnp.float32), ((0, 0), (0, pad)))
    hs_p = jnp.pad(hidden_states.astype(jnp.float32),
                   ((0, 0), (0, pad), (0, 0)))
    bc = jnp.concatenate(
        [bx_p, sc_p[:, :, None], lb_p[:, :, None]], axis=2)
    br = jnp.transpose(bc, (0, 2, 1))

    obox, oaux, ohs = pl.pallas_call(
        _body,
        grid=(_B,),
        in_specs=[
            pl.BlockSpec(memory_space=pltpu.SMEM),
            pl.BlockSpec((1, 6, _NP), lambda i: (i, 0, 0)),
            pl.BlockSpec((1, _NP, 6), lambda i: (i, 0, 0)),
            pl.BlockSpec((1, _NP, _D), lambda i: (i, 0, 0)),
        ],
        out_specs=[
            pl.BlockSpec((1, 32, 4), lambda i: (i, 0, 0)),
            pl.BlockSpec((1, 32, 4), lambda i: (i, 0, 0)),
            pl.BlockSpec((1, 32, _D), lambda i: (i, 0, 0)),
        ],
        out_shape=[
            jax.ShapeDtypeStruct((_B, 32, 4), jnp.float32),
            jax.ShapeDtypeStruct((_B, 32, 4), jnp.float32),
            jax.ShapeDtypeStruct((_B, 32, _D), jnp.float32),
        ],
        scratch_shapes=[
            pltpu.VMEM((_NP, _NP), jnp.int8),
            pltpu.VMEM((32, _NP), jnp.float32),
            pltpu.VMEM((1, _NP), jnp.int8),
        ],
    )(image_sizes.astype(jnp.float32), br, bc, hs_p)

    bxs = obox[:, :30, :]
    scs = oaux[:, :30, 0]
    lbs = oaux[:, :30, 1].astype(jnp.int32)
    msk = oaux[:, :30, 2] > 0.5
    hss = ohs[:, :30, :]
    return (bxs, scs, lbs, hss, msk)


# speedup vs baseline: 1.6555x; 1.6555x over previous
import jax
import jax.numpy as jnp
from jax import lax
from jax.experimental import pallas as pl
from jax.experimental.pallas import tpu as pltpu

_B, _N, _D = 4, 5000, 256
_NP = 5120
_JB = 256
_KB = 512
_IB = 1024
_NB = _NP // _JB
_NEG = -1e30
_IOU_T = 0.5
_TH = 0.2
_KMIN, _KMAX = 3, 15


def _body(sz_ref, br_ref, bc_ref, hs_ref, obox_ref, oaux_ref, ohs_ref,
          A_ref, OH_ref, KP_ref):
    pid = pl.program_id(0)
    szh = sz_ref[pid, 0]
    szw = sz_ref[pid, 1]

    x1c = bc_ref[0, :, 0:1]
    y1c = bc_ref[0, :, 1:2]
    x2c = bc_ref[0, :, 2:3]
    y2c = bc_ref[0, :, 3:4]
    scc = bc_ref[0, :, 4:5]
    lbc = bc_ref[0, :, 5:6]

    maxc = jnp.max(br_ref[0, 0:4, :]) + 1.0

    iotar = lax.broadcasted_iota(jnp.int32, (1, _NP), 1)

    def jloop(jb, _):
        j0 = pl.multiple_of(jb * _JB, _JB)
        blb = br_ref[0, 5:6, pl.ds(j0, _JB)]
        bx1 = br_ref[0, 0:1, pl.ds(j0, _JB)] + blb * maxc
        by1 = br_ref[0, 1:2, pl.ds(j0, _JB)] + blb * maxc
        bx2 = br_ref[0, 2:3, pl.ds(j0, _JB)] + blb * maxc
        by2 = br_ref[0, 3:4, pl.ds(j0, _JB)] + blb * maxc
        bar = jnp.maximum(bx2 - bx1, 0.0) * jnp.maximum(by2 - by1, 0.0)
        bsc = br_ref[0, 4:5, pl.ds(j0, _JB)]
        bio = j0 + lax.broadcasted_iota(jnp.int32, (1, _JB), 1)

        def iloop(ib, _2):
            i0 = pl.multiple_of(ib * _IB, _IB)
            clb = bc_ref[0, pl.ds(i0, _IB), 5:6]
            cx1 = bc_ref[0, pl.ds(i0, _IB), 0:1] + clb * maxc
            cy1 = bc_ref[0, pl.ds(i0, _IB), 1:2] + clb * maxc
            cx2 = bc_ref[0, pl.ds(i0, _IB), 2:3] + clb * maxc
            cy2 = bc_ref[0, pl.ds(i0, _IB), 3:4] + clb * maxc
            car = (jnp.maximum(cx2 - cx1, 0.0) *
                   jnp.maximum(cy2 - cy1, 0.0))
            csc = bc_ref[0, pl.ds(i0, _IB), 4:5]
            cio = i0 + lax.broadcasted_iota(jnp.int32, (_IB, 1), 0)
            iw = jnp.maximum(
                jnp.minimum(cx2, bx2) - jnp.maximum(cx1, bx1), 0.0)
            ih = jnp.maximum(
                jnp.minimum(cy2, by2) - jnp.maximum(cy1, by1), 0.0)
            inter = iw * ih
            iou = inter / (car + bar - inter + 1e-9)
            prio = (csc > bsc) | ((csc == bsc) & (cio < bio))
            sup = jnp.where((iou > _IOU_T) & prio, 1, 0).astype(jnp.int8)
            A_ref[pl.ds(i0, _IB), pl.ds(j0, _JB)] = sup
            return 0

        lax.fori_loop(0, _NP // _IB, iloop, 0)
        return 0

    lax.fori_loop(0, _NB, jloop, 0)

    def fp_cond(c):
        return c[1]

    def fp_body(c):
        keep, _ = c
        KP_ref[...] = keep.astype(jnp.int8)

        def acc(kb, s):
            k0 = pl.multiple_of(kb * _KB, _KB)
            return s + jnp.dot(KP_ref[0:1, pl.ds(k0, _KB)],
                               A_ref[pl.ds(k0, _KB), :],
                               preferred_element_type=jnp.int32)

        supn = lax.fori_loop(0, _NP // _KB, acc,
                             jnp.zeros((1, _NP), dtype=jnp.int32))
        keep_new = jnp.where(supn == 0, 1, 0)
        ndiff = jnp.sum(jnp.abs(keep_new - keep))
        return keep_new, ndiff > 0

    keep, _ = lax.while_loop(
        fp_cond, fp_body,
        (jnp.ones((1, _NP), dtype=jnp.int32), True))
    kb = keep != 0

    scr = br_ref[0, 4:5, :]
    lbr = br_ref[0, 5:6, :]
    is_h = lbr == 0.0
    h_sc0 = jnp.where(kb & is_h, scr, _NEG)
    o_sc0 = jnp.where(kb & (~is_h), scr, _NEG)
    n_h = jnp.sum(jnp.where(h_sc0 >= _TH, 1, 0))
    n_o = jnp.sum(jnp.where(o_sc0 >= _TH, 1, 0))
    k_h = jnp.clip(n_h, _KMIN, _KMAX)
    k_o = jnp.clip(n_o, _KMIN, _KMAX)

    OH_ref[...] = jnp.zeros((32, _NP), dtype=jnp.float32)
    rows32 = lax.broadcasted_iota(jnp.int32, (32, 1), 0)

    def pick(r, c):
        h, o, vvec = c
        cur = jnp.where(r < _KMAX, h, o)
        m = jnp.max(cur)
        selidx = jnp.min(jnp.where(cur == m, iotar, _NP))
        OH_ref[pl.ds(r, 1), :] = jnp.where(iotar == selidx, 1.0, 0.0)
        vvec = jnp.where(rows32 == r, m, vvec)
        hit = iotar == selidx
        h = jnp.where(hit & (r < _KMAX), -jnp.inf, h)
        o = jnp.where(hit & (r >= _KMAX), -jnp.inf, o)
        return h, o, vvec

    _, _, vvec = lax.fori_loop(
        0, 2 * _KMAX, pick,
        (h_sc0, o_sc0, jnp.full((32, 1), -jnp.inf, dtype=jnp.float32)))

    limit = jnp.where(rows32 < _KMAX, k_h, _KMAX + k_o)
    mask = (rows32 < limit) & (vvec > _NEG * 0.5)
    mf = jnp.where(mask, 1.0, 0.0)

    bxcol = jnp.concatenate(
        [jnp.maximum(x1c, 0.0), jnp.maximum(y1c, 0.0),
         jnp.minimum(x2c, szw), jnp.minimum(y2c, szh)], axis=1)
    oh = OH_ref[...]
    hp = lax.Precision.HIGHEST
    gbox = jnp.dot(oh, bxcol, precision=hp,
                   preferred_element_type=jnp.float32) * mf
    gsc = jnp.dot(oh, scc, precision=hp,
                  preferred_element_type=jnp.float32) * mf
    glb = jnp.dot(oh, lbc, precision=hp,
                  preferred_element_type=jnp.float32)
    ghs = jnp.dot(oh, hs_ref[0], precision=hp,
                  preferred_element_type=jnp.float32) * mf
    lbout = jnp.where(mask, glb, -1.0)

    obox_ref[0] = gbox
    oaux_ref[0] = jnp.concatenate(
        [gsc, lbout, mf, jnp.zeros((32, 1), dtype=jnp.float32)], axis=1)
    ohs_ref[0] = ghs


def kernel(boxes, scores, hidden_states, labels, image_sizes):
    pad = _NP - _N
    sc_p = jnp.pad(scores.astype(jnp.float32), ((0, 0), (0, pad)),
                   constant_values=_NEG)
    bx_p = jnp.pad(boxes.astype(jnp.float32), ((0, 0), (0, pad), (0, 0)))
    lb_p = jnp.pad(labels.astype(jnp.float32), ((0, 0), (0, pad)))
    hs_p = jnp.pad(hidden_states.astype(jnp.float32),
                   ((0, 0), (0, pad), (0, 0)))
    bc = jnp.concatenate(
        [bx_p, sc_p[:, :, None], lb_p[:, :, None]], axis=2)
    br = jnp.transpose(bc, (0, 2, 1))

    obox, oaux, ohs = pl.pallas_call(
        _body,
        grid=(_B,),
        in_specs=[
            pl.BlockSpec(memory_space=pltpu.SMEM),
            pl.BlockSpec((1, 6, _NP), lambda i: (i, 0, 0)),
            pl.BlockSpec((1, _NP, 6), lambda i: (i, 0, 0)),
            pl.BlockSpec((1, _NP, _D), lambda i: (i, 0, 0)),
        ],
        out_specs=[
            pl.BlockSpec((1, 32, 4), lambda i: (i, 0, 0)),
            pl.BlockSpec((1, 32, 4), lambda i: (i, 0, 0)),
            pl.BlockSpec((1, 32, _D), lambda i: (i, 0, 0)),
        ],
        out_shape=[
            jax.ShapeDtypeStruct((_B, 32, 4), jnp.float32),
            jax.ShapeDtypeStruct((_B, 32, 4), jnp.float32),
            jax.ShapeDtypeStruct((_B, 32, _D), jnp.float32),
        ],
        scratch_shapes=[
            pltpu.VMEM((_NP, _NP), jnp.int8),
            pltpu.VMEM((32, _NP), jnp.float32),
            pltpu.VMEM((1, _NP), jnp.int8),
        ],
    )(image_sizes.astype(jnp.float32), br, bc, hs_p)

    bxs = obox[:, :30, :]
    scs = oaux[:, :30, 0]
    lbs = oaux[:, :30, 1].astype(jnp.int32)
    msk = oaux[:, :30, 2] > 0.5
    hss = ohs[:, :30, :]
    return (bxs, scs, lbs, hss, msk)


# speedup vs baseline: 2.2256x; 1.3444x over previous
import jax
import jax.numpy as jnp
from jax import lax
from jax.experimental import pallas as pl
from jax.experimental.pallas import tpu as pltpu

_B, _N, _D = 4, 5000, 256
_NP = 5120
_JB = 512
_KB = 512
_IB = 1024
_NB = _NP // _JB
_NEG = -1e30
_IOU_T = 0.5
_TH = 0.2
_KMIN, _KMAX = 3, 15


def _body(sz_ref, br_ref, bc_ref, hs_ref, obox_ref, oaux_ref, ohs_ref,
          A_ref, OH_ref, KP_ref):
    pid = pl.program_id(0)
    szh = sz_ref[pid, 0]
    szw = sz_ref[pid, 1]

    x1c = bc_ref[0, :, 0:1]
    y1c = bc_ref[0, :, 1:2]
    x2c = bc_ref[0, :, 2:3]
    y2c = bc_ref[0, :, 3:4]
    scc = bc_ref[0, :, 4:5]
    lbc = bc_ref[0, :, 5:6]

    maxc = jnp.max(br_ref[0, 0:4, :]) + 1.0

    iotar = lax.broadcasted_iota(jnp.int32, (1, _NP), 1)

    def jloop(jb, _):
        j0 = pl.multiple_of(jb * _JB, _JB)
        blb = br_ref[0, 5:6, pl.ds(j0, _JB)]
        bx1 = br_ref[0, 0:1, pl.ds(j0, _JB)] + blb * maxc
        by1 = br_ref[0, 1:2, pl.ds(j0, _JB)] + blb * maxc
        bx2 = br_ref[0, 2:3, pl.ds(j0, _JB)] + blb * maxc
        by2 = br_ref[0, 3:4, pl.ds(j0, _JB)] + blb * maxc
        bar = jnp.maximum(bx2 - bx1, 0.0) * jnp.maximum(by2 - by1, 0.0)
        bsc = br_ref[0, 4:5, pl.ds(j0, _JB)]
        bio = j0 + lax.broadcasted_iota(jnp.int32, (1, _JB), 1)

        def iloop(ib, _2):
            i0 = pl.multiple_of(ib * _IB, _IB)
            clb = bc_ref[0, pl.ds(i0, _IB), 5:6]
            cx1 = bc_ref[0, pl.ds(i0, _IB), 0:1] + clb * maxc
            cy1 = bc_ref[0, pl.ds(i0, _IB), 1:2] + clb * maxc
            cx2 = bc_ref[0, pl.ds(i0, _IB), 2:3] + clb * maxc
            cy2 = bc_ref[0, pl.ds(i0, _IB), 3:4] + clb * maxc
            car = (jnp.maximum(cx2 - cx1, 0.0) *
                   jnp.maximum(cy2 - cy1, 0.0))
            csc = bc_ref[0, pl.ds(i0, _IB), 4:5]
            cio = i0 + lax.broadcasted_iota(jnp.int32, (_IB, 1), 0)
            iw = jnp.maximum(
                jnp.minimum(cx2, bx2) - jnp.maximum(cx1, bx1), 0.0)
            ih = jnp.maximum(
                jnp.minimum(cy2, by2) - jnp.maximum(cy1, by1), 0.0)
            inter = iw * ih
            iou = inter / (car + bar - inter + 1e-9)
            prio = (csc > bsc) | ((csc == bsc) & (cio < bio))
            sup = jnp.where((iou > _IOU_T) & prio, 1, 0).astype(jnp.int8)
            A_ref[pl.ds(i0, _IB), pl.ds(j0, _JB)] = sup
            return 0

        lax.fori_loop(0, _NP // _IB, iloop, 0)
        return 0

    lax.fori_loop(0, _NB, jloop, 0)

    def fp_cond(c):
        return c[1]

    def fp_body(c):
        keep, _ = c
        KP_ref[...] = keep.astype(jnp.int8)

        def acc(kb, s):
            k0 = pl.multiple_of(kb * _KB, _KB)
            return s + jnp.dot(KP_ref[0:1, pl.ds(k0, _KB)],
                               A_ref[pl.ds(k0, _KB), :],
                               preferred_element_type=jnp.int32)

        supn = lax.fori_loop(0, _NP // _KB, acc,
                             jnp.zeros((1, _NP), dtype=jnp.int32))
        keep_new = jnp.where(supn == 0, 1, 0)
        ndiff = jnp.sum(jnp.abs(keep_new - keep))
        return keep_new, ndiff > 0

    keep, _ = lax.while_loop(
        fp_cond, fp_body,
        (jnp.ones((1, _NP), dtype=jnp.int32), True))
    kb = keep != 0

    scr = br_ref[0, 4:5, :]
    lbr = br_ref[0, 5:6, :]
    is_h = lbr == 0.0
    h_sc0 = jnp.where(kb & is_h, scr, _NEG)
    o_sc0 = jnp.where(kb & (~is_h), scr, _NEG)
    n_h = jnp.sum(jnp.where(h_sc0 >= _TH, 1, 0))
    n_o = jnp.sum(jnp.where(o_sc0 >= _TH, 1, 0))
    k_h = jnp.clip(n_h, _KMIN, _KMAX)
    k_o = jnp.clip(n_o, _KMIN, _KMAX)

    OH_ref[...] = jnp.zeros((32, _NP), dtype=jnp.float32)
    rows32 = lax.broadcasted_iota(jnp.int32, (32, 1), 0)

    def pick(r, c):
        h, o, vvec = c
        cur = jnp.where(r < _KMAX, h, o)
        m = jnp.max(cur)
        selidx = jnp.min(jnp.where(cur == m, iotar, _NP))
        OH_ref[pl.ds(r, 1), :] = jnp.where(iotar == selidx, 1.0, 0.0)
        vvec = jnp.where(rows32 == r, m, vvec)
        hit = iotar == selidx
        h = jnp.where(hit & (r < _KMAX), -jnp.inf, h)
        o = jnp.where(hit & (r >= _KMAX), -jnp.inf, o)
        return h, o, vvec

    _, _, vvec = lax.fori_loop(
        0, 2 * _KMAX, pick,
        (h_sc0, o_sc0, jnp.full((32, 1), -jnp.inf, dtype=jnp.float32)))

    limit = jnp.where(rows32 < _KMAX, k_h, _KMAX + k_o)
    mask = (rows32 < limit) & (vvec > _NEG * 0.5)
    mf = jnp.where(mask, 1.0, 0.0)

    bxcol = jnp.concatenate(
        [jnp.maximum(x1c, 0.0), jnp.maximum(y1c, 0.0),
         jnp.minimum(x2c, szw), jnp.minimum(y2c, szh)], axis=1)
    oh = OH_ref[...]
    hp = lax.Precision.HIGHEST
    gbox = jnp.dot(oh, bxcol, precision=hp,
                   preferred_element_type=jnp.float32) * mf
    gsc = jnp.dot(oh, scc, precision=hp,
                  preferred_element_type=jnp.float32) * mf
    glb = jnp.dot(oh, lbc, precision=hp,
                  preferred_element_type=jnp.float32)
    ghs = jnp.dot(oh, hs_ref[0], precision=hp,
                  preferred_element_type=jnp.float32) * mf
    lbout = jnp.where(mask, glb, -1.0)

    obox_ref[0] = gbox
    oaux_ref[0] = jnp.concatenate(
        [gsc, lbout, mf, jnp.zeros((32, 1), dtype=jnp.float32)], axis=1)
    ohs_ref[0] = ghs


def kernel(boxes, scores, hidden_states, labels, image_sizes):
    pad = _NP - _N
    sc_p = jnp.pad(scores.astype(jnp.float32), ((0, 0), (0, pad)),
                   constant_values=_NEG)
    bx_p = jnp.pad(boxes.astype(jnp.float32), ((0, 0), (0, pad), (0, 0)))
    lb_p = jnp.pad(labels.astype(jnp.float32), ((0, 0), (0, pad)))
    hs_p = jnp.pad(hidden_states.astype(jnp.float32),
                   ((0, 0), (0, pad), (0, 0)))
    bc = jnp.concatenate(
        [bx_p, sc_p[:, :, None], lb_p[:, :, None]], axis=2)
    br = jnp.transpose(bc, (0, 2, 1))

    obox, oaux, ohs = pl.pallas_call(
        _body,
        grid=(_B,),
        in_specs=[
            pl.BlockSpec(memory_space=pltpu.SMEM),
            pl.BlockSpec((1, 6, _NP), lambda i: (i, 0, 0)),
            pl.BlockSpec((1, _NP, 6), lambda i: (i, 0, 0)),
            pl.BlockSpec((1, _NP, _D), lambda i: (i, 0, 0)),
        ],
        out_specs=[
            pl.BlockSpec((1, 32, 4), lambda i: (i, 0, 0)),
            pl.BlockSpec((1, 32, 4), lambda i: (i, 0, 0)),
            pl.BlockSpec((1, 32, _D), lambda i: (i, 0, 0)),
        ],
        out_shape=[
            jax.ShapeDtypeStruct((_B, 32, 4), jnp.float32),
            jax.ShapeDtypeStruct((_B, 32, 4), jnp.float32),
            jax.ShapeDtypeStruct((_B, 32, _D), jnp.float32),
        ],
        scratch_shapes=[
            pltpu.VMEM((_NP, _NP), jnp.int8),
            pltpu.VMEM((32, _NP), jnp.float32),
            pltpu.VMEM((1, _NP), jnp.int8),
        ],
    )(image_sizes.astype(jnp.float32), br, bc, hs_p)

    bxs = obox[:, :30, :]
    scs = oaux[:, :30, 0]
    lbs = oaux[:, :30, 1].astype(jnp.int32)
    msk = oaux[:, :30, 2] > 0.5
    hss = ohs[:, :30, :]
    return (bxs, scs, lbs, hss, msk)


# speedup vs baseline: 2.5069x; 1.1264x over previous
import jax
import jax.numpy as jnp
from jax import lax
from jax.experimental import pallas as pl
from jax.experimental.pallas import tpu as pltpu

_B, _N, _D = 4, 5000, 256
_NP = 5120
_JB = 1024
_KB = 512
_IB = 1024
_NB = _NP // _JB
_NEG = -1e30
_IOU_T = 0.5
_TH = 0.2
_KMIN, _KMAX = 3, 15


def _body(sz_ref, br_ref, bc_ref, hs_ref, obox_ref, oaux_ref, ohs_ref,
          A_ref, OH_ref, KP_ref):
    pid = pl.program_id(0)
    szh = sz_ref[pid, 0]
    szw = sz_ref[pid, 1]

    x1c = bc_ref[0, :, 0:1]
    y1c = bc_ref[0, :, 1:2]
    x2c = bc_ref[0, :, 2:3]
    y2c = bc_ref[0, :, 3:4]
    scc = bc_ref[0, :, 4:5]
    lbc = bc_ref[0, :, 5:6]

    maxc = jnp.max(br_ref[0, 0:4, :]) + 1.0

    iotar = lax.broadcasted_iota(jnp.int32, (1, _NP), 1)

    def jloop(jb, _):
        j0 = pl.multiple_of(jb * _JB, _JB)
        blb = br_ref[0, 5:6, pl.ds(j0, _JB)]
        bx1 = br_ref[0, 0:1, pl.ds(j0, _JB)] + blb * maxc
        by1 = br_ref[0, 1:2, pl.ds(j0, _JB)] + blb * maxc
        bx2 = br_ref[0, 2:3, pl.ds(j0, _JB)] + blb * maxc
        by2 = br_ref[0, 3:4, pl.ds(j0, _JB)] + blb * maxc
        bar = jnp.maximum(bx2 - bx1, 0.0) * jnp.maximum(by2 - by1, 0.0)
        bsc = br_ref[0, 4:5, pl.ds(j0, _JB)]
        bio = j0 + lax.broadcasted_iota(jnp.int32, (1, _JB), 1)

        def iloop(ib, _2):
            i0 = pl.multiple_of(ib * _IB, _IB)
            clb = bc_ref[0, pl.ds(i0, _IB), 5:6]
            cx1 = bc_ref[0, pl.ds(i0, _IB), 0:1] + clb * maxc
            cy1 = bc_ref[0, pl.ds(i0, _IB), 1:2] + clb * maxc
            cx2 = bc_ref[0, pl.ds(i0, _IB), 2:3] + clb * maxc
            cy2 = bc_ref[0, pl.ds(i0, _IB), 3:4] + clb * maxc
            car = (jnp.maximum(cx2 - cx1, 0.0) *
                   jnp.maximum(cy2 - cy1, 0.0))
            csc = bc_ref[0, pl.ds(i0, _IB), 4:5]
            cio = i0 + lax.broadcasted_iota(jnp.int32, (_IB, 1), 0)
            iw = jnp.maximum(
                jnp.minimum(cx2, bx2) - jnp.maximum(cx1, bx1), 0.0)
            ih = jnp.maximum(
                jnp.minimum(cy2, by2) - jnp.maximum(cy1, by1), 0.0)
            inter = iw * ih
            iou = inter / (car + bar - inter + 1e-9)
            prio = (csc > bsc) | ((csc == bsc) & (cio < bio))
            sup = jnp.where((iou > _IOU_T) & prio, 1, 0).astype(jnp.int8)
            A_ref[pl.ds(i0, _IB), pl.ds(j0, _JB)] = sup
            return 0

        lax.fori_loop(0, _NP // _IB, iloop, 0)
        return 0

    lax.fori_loop(0, _NB, jloop, 0)

    def fp_cond(c):
        return c[1]

    def fp_body(c):
        keep, _ = c
        KP_ref[...] = keep.astype(jnp.int8)

        def acc(kb, s):
            k0 = pl.multiple_of(kb * _KB, _KB)
            return s + jnp.dot(KP_ref[0:1, pl.ds(k0, _KB)],
                               A_ref[pl.ds(k0, _KB), :],
                               preferred_element_type=jnp.int32)

        supn = lax.fori_loop(0, _NP // _KB, acc,
                             jnp.zeros((1, _NP), dtype=jnp.int32))
        keep_new = jnp.where(supn == 0, 1, 0)
        ndiff = jnp.sum(jnp.abs(keep_new - keep))
        return keep_new, ndiff > 0

    keep, _ = lax.while_loop(
        fp_cond, fp_body,
        (jnp.ones((1, _NP), dtype=jnp.int32), True))
    kb = keep != 0

    scr = br_ref[0, 4:5, :]
    lbr = br_ref[0, 5:6, :]
    is_h = lbr == 0.0
    h_sc0 = jnp.where(kb & is_h, scr, _NEG)
    o_sc0 = jnp.where(kb & (~is_h), scr, _NEG)
    n_h = jnp.sum(jnp.where(h_sc0 >= _TH, 1, 0))
    n_o = jnp.sum(jnp.where(o_sc0 >= _TH, 1, 0))
    k_h = jnp.clip(n_h, _KMIN, _KMAX)
    k_o = jnp.clip(n_o, _KMIN, _KMAX)

    OH_ref[...] = jnp.zeros((32, _NP), dtype=jnp.float32)
    rows32 = lax.broadcasted_iota(jnp.int32, (32, 1), 0)

    def pick(r, c):
        h, o, vvec = c
        cur = jnp.where(r < _KMAX, h, o)
        m = jnp.max(cur)
        selidx = jnp.min(jnp.where(cur == m, iotar, _NP))
        OH_ref[pl.ds(r, 1), :] = jnp.where(iotar == selidx, 1.0, 0.0)
        vvec = jnp.where(rows32 == r, m, vvec)
        hit = iotar == selidx
        h = jnp.where(hit & (r < _KMAX), -jnp.inf, h)
        o = jnp.where(hit & (r >= _KMAX), -jnp.inf, o)
        return h, o, vvec

    _, _, vvec = lax.fori_loop(
        0, 2 * _KMAX, pick,
        (h_sc0, o_sc0, jnp.full((32, 1), -jnp.inf, dtype=jnp.float32)))

    limit = jnp.where(rows32 < _KMAX, k_h, _KMAX + k_o)
    mask = (rows32 < limit) & (vvec > _NEG * 0.5)
    mf = jnp.where(mask, 1.0, 0.0)

    bxcol = jnp.concatenate(
        [jnp.maximum(x1c, 0.0), jnp.maximum(y1c, 0.0),
         jnp.minimum(x2c, szw), jnp.minimum(y2c, szh)], axis=1)
    oh = OH_ref[...]
    hp = lax.Precision.HIGHEST
    gbox = jnp.dot(oh, bxcol, precision=hp,
                   preferred_element_type=jnp.float32) * mf
    gsc = jnp.dot(oh, scc, precision=hp,
                  preferred_element_type=jnp.float32) * mf
    glb = jnp.dot(oh, lbc, precision=hp,
                  preferred_element_type=jnp.float32)
    ghs = jnp.dot(oh, hs_ref[0], precision=hp,
                  preferred_element_type=jnp.float32) * mf
    lbout = jnp.where(mask, glb, -1.0)

    obox_ref[0] = gbox
    oaux_ref[0] = jnp.concatenate(
        [gsc, lbout, mf, jnp.zeros((32, 1), dtype=jnp.float32)], axis=1)
    ohs_ref[0] = ghs


def kernel(boxes, scores, hidden_states, labels, image_sizes):
    pad = _NP - _N
    sc_p = jnp.pad(scores.astype(jnp.float32), ((0, 0), (0, pad)),
                   constant_values=_NEG)
    bx_p = jnp.pad(boxes.astype(jnp.float32), ((0, 0), (0, pad), (0, 0)))
    lb_p = jnp.pad(labels.astype(jnp.float32), ((0, 0), (0, pad)))
    hs_p = jnp.pad(hidden_states.astype(jnp.float32),
                   ((0, 0), (0, pad), (0, 0)))
    bc = jnp.concatenate(
        [bx_p, sc_p[:, :, None], lb_p[:, :, None]], axis=2)
    br = jnp.transpose(bc, (0, 2, 1))

    obox, oaux, ohs = pl.pallas_call(
        _body,
        grid=(_B,),
        in_specs=[
            pl.BlockSpec(memory_space=pltpu.SMEM),
            pl.BlockSpec((1, 6, _NP), lambda i: (i, 0, 0)),
            pl.BlockSpec((1, _NP, 6), lambda i: (i, 0, 0)),
            pl.BlockSpec((1, _NP, _D), lambda i: (i, 0, 0)),
        ],
        out_specs=[
            pl.BlockSpec((1, 32, 4), lambda i: (i, 0, 0)),
            pl.BlockSpec((1, 32, 4), lambda i: (i, 0, 0)),
            pl.BlockSpec((1, 32, _D), lambda i: (i, 0, 0)),
        ],
        out_shape=[
            jax.ShapeDtypeStruct((_B, 32, 4), jnp.float32),
            jax.ShapeDtypeStruct((_B, 32, 4), jnp.float32),
            jax.ShapeDtypeStruct((_B, 32, _D), jnp.float32),
        ],
        scratch_shapes=[
            pltpu.VMEM((_NP, _NP), jnp.int8),
            pltpu.VMEM((32, _NP), jnp.float32),
            pltpu.VMEM((1, _NP), jnp.int8),
        ],
    )(image_sizes.astype(jnp.float32), br, bc, hs_p)

    bxs = obox[:, :30, :]
    scs = oaux[:, :30, 0]
    lbs = oaux[:, :30, 1].astype(jnp.int32)
    msk = oaux[:, :30, 2] > 0.5
    hss = ohs[:, :30, :]
    return (bxs, scs, lbs, hss, msk)
